# Initial kernel scaffold; baseline (speedup 1.0000x reference)
import jax
import jax.numpy as jnp
from jax.experimental import pallas as pl


def kernel(mel, pitch_embed, W, b):
    raise NotImplementedError("write your pallas kernel here")


# speedup vs baseline: 1.8416x; 1.8416x over previous
import functools

import jax
import jax.numpy as jnp
from jax import lax
from jax.experimental import pallas as pl
from jax.experimental.pallas import tpu as pltpu
from jax.experimental.pallas import tpu_sc as plsc


def _project_table(pitch_embed, W, b):
    V, H = pitch_embed.shape
    O = W.shape[0]

    def body(e_ref, w_ref, b_ref, o_ref):
        o_ref[...] = lax.dot_general(
            e_ref[...], w_ref[...],
            dimension_numbers=(((1,), (1,)), ((), ())),
            preferred_element_type=jnp.float32,
        ) + b_ref[...]

    return pl.pallas_call(
        body,
        out_shape=jax.ShapeDtypeStruct((V, O), jnp.float32),
    )(pitch_embed, W, b.reshape(1, O))


@functools.lru_cache(maxsize=None)
def _make_sc_gather(V, O, NW, NC, C, L):
    mesh = plsc.VectorSubcoreMesh(core_axis_name="c", subcore_axis_name="s")

    @functools.partial(
        pl.kernel,
        out_type=jax.ShapeDtypeStruct((NW, C, L, O), jnp.float32),
        mesh=mesh,
        scratch_types=[
            pltpu.VMEM((C, L), jnp.int32),
            pltpu.VMEM((2, L, O), jnp.float32),
            pltpu.SemaphoreType.DMA,
            pltpu.SemaphoreType.DMA,
            pltpu.SemaphoreType.DMA,
            pltpu.SemaphoreType.DMA,
        ],
        compiler_params=pltpu.CompilerParams(use_tc_tiling_on_sc=False),
    )
    def k(table_hbm, idx_hbm, out_hbm, idx_v, rows_v, g0, g1, s0, s1):
        gsem = (g0, g1)
        ssem = (s0, s1)
        wid = lax.axis_index("s") * NC + lax.axis_index("c")
        pltpu.sync_copy(idx_hbm.at[wid], idx_v)
        gathers = [None, None]
        stores = [None, None]
        gathers[0] = pltpu.async_copy(
            table_hbm.at[idx_v.at[0]], rows_v.at[0], gsem[0])
        for j in range(C):
            cur = j % 2
            nxt = 1 - cur
            gathers[cur].wait()
            if j + 1 < C:
                if stores[nxt] is not None:
                    stores[nxt].wait()
                    stores[nxt] = None
                gathers[nxt] = pltpu.async_copy(
                    table_hbm.at[idx_v.at[j + 1]], rows_v.at[nxt], gsem[nxt])
            stores[cur] = pltpu.async_copy(
                rows_v.at[cur], out_hbm.at[wid, j], ssem[cur])
        for st in stores:
            if st is not None:
                st.wait()

    return k


def kernel(mel, pitch_embed, W, b):
    B, T = mel.shape
    V, H = pitch_embed.shape
    O = W.shape[0]

    info = plsc.get_sparse_core_info()
    NC, NS, L = info.num_cores, info.num_subcores, info.num_lanes
    NW = NC * NS
    tokens = B * T
    assert tokens % (NW * 128) == 0
    C = tokens // (NW * 128)

    proj = _project_table(pitch_embed, W, b)
    idx = mel.reshape(NW, C, 128).astype(jnp.int32)
    out = _make_sc_gather(V, O, NW, NC, C, 128)(proj, idx)
    return out.reshape(B, T, O)


# speedup vs baseline: 2.0148x; 1.0941x over previous
import functools

import jax
import jax.numpy as jnp
from jax import lax
from jax.experimental import pallas as pl
from jax.experimental.pallas import tpu as pltpu
from jax.experimental.pallas import tpu_sc as plsc

_LANES = 128


def _project_table(pitch_embed, W, b):
    V, H = pitch_embed.shape
    O = W.shape[0]
    Wp = jnp.pad(W, ((0, _LANES - O), (0, 0)))
    bp = jnp.pad(b, (0, _LANES - O))

    def body(e_ref, w_ref, b_ref, o_ref):
        o_ref[...] = lax.dot_general(
            e_ref[...], w_ref[...],
            dimension_numbers=(((1,), (1,)), ((), ())),
            preferred_element_type=jnp.float32,
        ) + b_ref[...]

    return pl.pallas_call(
        body,
        out_shape=jax.ShapeDtypeStruct((V, _LANES), jnp.float32),
    )(pitch_embed, Wp, bp.reshape(1, _LANES))


@functools.lru_cache(maxsize=None)
def _make_sc_gather(V, O, NW, NC, C):
    mesh = plsc.VectorSubcoreMesh(core_axis_name="c", subcore_axis_name="s")

    @functools.partial(
        pl.kernel,
        out_type=jax.ShapeDtypeStruct((NW * C * 128, _LANES), jnp.float32),
        mesh=mesh,
        scratch_types=[
            pltpu.VMEM((C, 128), jnp.int32),
            pltpu.VMEM((2, 128, _LANES), jnp.float32),
            pltpu.SemaphoreType.DMA,
            pltpu.SemaphoreType.DMA,
            pltpu.SemaphoreType.DMA,
            pltpu.SemaphoreType.DMA,
        ],
    )
    def k(table_hbm, idx_hbm, out_hbm, idx_v, rows_v, g0, g1, s0, s1):
        gsem = (g0, g1)
        ssem = (s0, s1)
        wid = lax.axis_index("s") * NC + lax.axis_index("c")
        base = wid * C * 128
        pltpu.sync_copy(idx_hbm.at[pl.ds(wid * C, C)], idx_v)
        gathers = [None, None]
        stores = [None, None]
        gathers[0] = pltpu.async_copy(
            table_hbm.at[idx_v.at[0]], rows_v.at[0], gsem[0])
        for j in range(C):
            cur = j % 2
            nxt = 1 - cur
            gathers[cur].wait()
            if j + 1 < C:
                if stores[nxt] is not None:
                    stores[nxt].wait()
                    stores[nxt] = None
                gathers[nxt] = pltpu.async_copy(
                    table_hbm.at[idx_v.at[j + 1]], rows_v.at[nxt], gsem[nxt])
            stores[cur] = pltpu.async_copy(
                rows_v.at[cur],
                out_hbm.at[pl.ds(base + j * 128, 128)], ssem[cur])
        for st in stores:
            if st is not None:
                st.wait()

    return k


def kernel(mel, pitch_embed, W, b):
    B, T = mel.shape
    V, H = pitch_embed.shape
    O = W.shape[0]

    info = plsc.get_sparse_core_info()
    NC, NS = info.num_cores, info.num_subcores
    NW = NC * NS
    tokens = B * T
    assert tokens % (NW * 128) == 0
    C = tokens // (NW * 128)

    proj = _project_table(pitch_embed, W, b)
    idx = mel.reshape(tokens // 128, 128).astype(jnp.int32)
    out = _make_sc_gather(V, O, NW, NC, C)(proj, idx)
    return out[:, :O].reshape(B, T, O)


# speedup vs baseline: 2.1624x; 1.0732x over previous
import functools

import jax
import jax.numpy as jnp
from jax import lax
from jax.experimental import pallas as pl
from jax.experimental.pallas import tpu as pltpu
from jax.experimental.pallas import tpu_sc as plsc

_LANES = 128
_NBUF = 4


def _project_table(pitch_embed, W, b):
    V, H = pitch_embed.shape
    O = W.shape[0]

    def body(e_ref, w_ref, b_ref, o_ref):
        res = lax.dot_general(
            e_ref[...], w_ref[...],
            dimension_numbers=(((1,), (1,)), ((), ())),
            preferred_element_type=jnp.float32,
        ) + b_ref[...]
        o_ref[...] = jnp.concatenate(
            [res, jnp.zeros((V, _LANES - O), jnp.float32)], axis=1)

    return pl.pallas_call(
        body,
        out_shape=jax.ShapeDtypeStruct((V, _LANES), jnp.float32),
    )(pitch_embed, W, b.reshape(1, O))


@functools.lru_cache(maxsize=None)
def _make_sc_gather(V, NW, NC, C):
    mesh = plsc.VectorSubcoreMesh(core_axis_name="c", subcore_axis_name="s")

    @functools.partial(
        pl.kernel,
        out_type=jax.ShapeDtypeStruct((NW * C * 128, _LANES), jnp.float32),
        mesh=mesh,
        scratch_types=[
            pltpu.VMEM((C, 128), jnp.int32),
            pltpu.VMEM((_NBUF, 128, _LANES), jnp.float32),
        ] + [pltpu.SemaphoreType.DMA] * (2 * _NBUF),
    )
    def k(table_hbm, idx_hbm, out_hbm, idx_v, rows_v, *sems):
        gsem = sems[:_NBUF]
        ssem = sems[_NBUF:]
        wid = lax.axis_index("s") * NC + lax.axis_index("c")
        base = wid * C * 128
        pltpu.sync_copy(idx_hbm.at[pl.ds(wid * C, C)], idx_v)
        gathers = [None] * _NBUF
        stores = [None] * _NBUF
        for j in range(min(_NBUF, C)):
            gathers[j] = pltpu.async_copy(
                table_hbm.at[idx_v.at[j]], rows_v.at[j], gsem[j])
        for j in range(C):
            buf = j % _NBUF
            gathers[buf].wait()
            stores[buf] = pltpu.async_copy(
                rows_v.at[buf],
                out_hbm.at[pl.ds(base + j * 128, 128)], ssem[buf])
            nj = j + _NBUF
            if nj < C:
                stores[buf].wait()
                stores[buf] = None
                gathers[buf] = pltpu.async_copy(
                    table_hbm.at[idx_v.at[nj]], rows_v.at[buf], gsem[buf])
        for st in stores:
            if st is not None:
                st.wait()

    return k


def kernel(mel, pitch_embed, W, b):
    B, T = mel.shape
    V, H = pitch_embed.shape
    O = W.shape[0]

    info = plsc.get_sparse_core_info()
    NC, NS = info.num_cores, info.num_subcores
    NW = NC * NS
    tokens = B * T
    assert tokens % (NW * 128) == 0
    C = tokens // (NW * 128)

    proj = _project_table(pitch_embed, W, b)
    idx = mel.reshape(tokens // 128, 128).astype(jnp.int32)
    out = _make_sc_gather(V, NW, NC, C)(proj, idx)
    return out[:, :O].reshape(B, T, O)
